# baseline (speedup 1.0000x reference)
import jax
import jax.numpy as jnp
from jax import lax
from jax.experimental import pallas as pl
from jax.experimental.pallas import tpu as pltpu
from jax.experimental.pallas import tpu_sc as plsc

WEIGHT_DECAY_ = 0.01

NC = 2
NS = 16
L = 16
NW = NC * NS
B = 16384
D = 64
PER_W = B // NW
CHUNK = 128
NCHUNK = PER_W // CHUNK


def _sc_body(u_hbm, i_hbm, j_hbm, w_hbm, h_hbm, p_hbm, sq_hbm,
             ui_v, ii_v, ji_v, ue_v, ie_v, je_v, pb_v, sq_v, sem):
    wid = lax.axis_index("s") * NC + lax.axis_index("c")

    pltpu.sync_copy(u_hbm.at[wid], ui_v)
    pltpu.sync_copy(i_hbm.at[wid], ii_v)
    pltpu.sync_copy(j_hbm.at[wid], ji_v)

    copies = []
    for tab, idx, dst in ((w_hbm, ui_v, ue_v), (h_hbm, ii_v, ie_v),
                          (h_hbm, ji_v, je_v)):
        for c in range(NCHUNK):
            copies.append(pltpu.async_copy(
                tab.at[idx.at[c]], dst.at[pl.ds(c * CHUNK, CHUNK)], sem))
    for cp in copies:
        cp.wait()

    def body1(h, sq_acc):
        for k in range(4):
            s = h * 4 + k
            p = jnp.zeros((L,), jnp.float32)
            for l in range(D // L):
                sl = pl.ds(l * L, L)
                ue = ue_v[s, sl]
                ie = ie_v[s, sl]
                je = je_v[s, sl]
                p = p + ue * (ie - je)
                sq_acc = sq_acc + (ue * ue + ie * ie + je * je)
            pb_v[s, pl.ds(0, L)] = p
        return sq_acc

    sq_acc = lax.fori_loop(0, PER_W // 4, body1, jnp.zeros((L,), jnp.float32))
    sq_v[...] = sq_acc

    pltpu.sync_copy(pb_v, p_hbm.at[wid])
    pltpu.sync_copy(sq_v, sq_hbm.at[wid])


def _tc_body(p_ref, s_ref, sq_ref, o_ref):
    x = lax.dot_general(p_ref[...], s_ref[...], (((1,), (0,)), ((), ())),
                        precision=lax.Precision.HIGHEST,
                        preferred_element_type=jnp.float32)
    ls = jnp.where(x >= 0.0,
                   -jnp.log1p(jnp.exp(-x)),
                   x - jnp.log1p(jnp.exp(x)))
    col = lax.broadcasted_iota(jnp.int32, ls.shape, 1)
    ls = jnp.where(col < 8, ls, 0.0)
    o_ref[0, 0] = WEIGHT_DECAY_ * jnp.sum(sq_ref[...]) - jnp.sum(ls)


@jax.jit
def kernel(u, i, j, W, H):
    u3 = u.astype(jnp.int32).reshape(NW, NCHUNK, CHUNK)
    i3 = i.astype(jnp.int32).reshape(NW, NCHUNK, CHUNK)
    j3 = j.astype(jnp.int32).reshape(NW, NCHUNK, CHUNK)

    mesh = plsc.VectorSubcoreMesh(core_axis_name="c", subcore_axis_name="s",
                                  num_cores=NC, num_subcores=NS)
    sc = pl.kernel(
        _sc_body,
        out_type=(jax.ShapeDtypeStruct((NW, PER_W, L), jnp.float32),
                  jax.ShapeDtypeStruct((NW, L), jnp.float32)),
        mesh=mesh,
        compiler_params=pltpu.CompilerParams(use_tc_tiling_on_sc=False),
        scratch_types=[
            pltpu.VMEM((NCHUNK, CHUNK), jnp.int32),
            pltpu.VMEM((NCHUNK, CHUNK), jnp.int32),
            pltpu.VMEM((NCHUNK, CHUNK), jnp.int32),
            pltpu.VMEM((PER_W, D), jnp.float32),
            pltpu.VMEM((PER_W, D), jnp.float32),
            pltpu.VMEM((PER_W, D), jnp.float32),
            pltpu.VMEM((PER_W, L), jnp.float32),
            pltpu.VMEM((L,), jnp.float32),
            pltpu.SemaphoreType.DMA,
        ],
    )
    p, sq = sc(u3, i3, j3, W, H)

    sel = (jnp.arange(128)[:, None] // L ==
           jnp.arange(128)[None, :]).astype(jnp.float32)

    loss = pl.pallas_call(
        _tc_body,
        out_shape=jax.ShapeDtypeStruct((1, 1), jnp.float32),
        out_specs=pl.BlockSpec(memory_space=pltpu.SMEM),
    )(p.reshape(B // 8, 8 * L), sel, sq.reshape(NW * L // 128, 128))
    return loss[0, 0]
